# baseline (speedup 1.0000x reference)
import functools

import jax
import jax.numpy as jnp
from jax import lax
from jax.experimental import pallas as pl
from jax.experimental.pallas import tpu as pltpu
from jax.experimental.pallas import tpu_sc as plsc

NUM_LAYERS = 24
STM_SIZE = 2048
EMBED_DIM = 1024

_INFO = plsc.get_sparse_core_info()
_NC = _INFO.num_cores
_NS = _INFO.num_subcores
_NW = _NC * _NS
_ROWS_PER_W = STM_SIZE // _NW


def _sc_gather_rows(mem_flat, rows):
    mesh = plsc.VectorSubcoreMesh(core_axis_name="c", subcore_axis_name="s")

    @functools.partial(
        pl.kernel,
        mesh=mesh,
        out_type=jax.ShapeDtypeStruct((STM_SIZE, EMBED_DIM), jnp.float32),
        scratch_types=[
            pltpu.VMEM((_ROWS_PER_W,), jnp.int32),
            pltpu.VMEM((_ROWS_PER_W, EMBED_DIM), jnp.float32),
            pltpu.SemaphoreType.DMA,
        ],
    )
    def body(mem_hbm, rows_hbm, out_hbm, idx_v, buf_v, sem):
        wid = lax.axis_index("s") * _NC + lax.axis_index("c")
        base = wid * _ROWS_PER_W
        pltpu.sync_copy(rows_hbm.at[pl.ds(base, _ROWS_PER_W)], idx_v)
        pltpu.async_copy(mem_hbm.at[idx_v], buf_v, sem).wait()
        pltpu.sync_copy(buf_v, out_hbm.at[pl.ds(base, _ROWS_PER_W)])

    return body(mem_flat, rows)


def kernel(memory, layer):
    mem_flat = memory.reshape(NUM_LAYERS * STM_SIZE, EMBED_DIM)
    rows = (jnp.asarray(layer, jnp.int32) * STM_SIZE
            + lax.iota(jnp.int32, STM_SIZE))
    out = _sc_gather_rows(mem_flat, rows)
    return out.reshape(1, STM_SIZE, EMBED_DIM)
